# baseline (speedup 1.0000x reference)
import functools

import jax
import jax.numpy as jnp
from jax import lax
from jax.experimental import pallas as pl
from jax.experimental.pallas import tpu as pltpu
from jax.experimental.pallas import tpu_sc as plsc

N_NODES = 10000
N_EDGES = 320000
D = 128

NC = 2
NS = 16
CHUNK = 16
G = 4
K = 1256
NGRP = K // G
NBUF = 2
E_PAD = NS * K * CHUNK

X_ROWS = 10000
XSLAB = 632
HALF = 5120
SLAB = HALF // NS
BM = 1000


def _final_body(x_ref, ws_ref, wn_ref, b_ref, agg_ref, o_ref):
    h = lax.dot_general(
        x_ref[...], ws_ref[...], (((1,), (1,)), ((), ())),
        preferred_element_type=jnp.float32)
    nei = lax.dot_general(
        agg_ref[...], wn_ref[...], (((1,), (1,)), ((), ())),
        preferred_element_type=jnp.float32)
    o_ref[...] = h + b_ref[...] + nei


def _final(x, W_self, W_nei, b_row, agg):
    return pl.pallas_call(
        _final_body,
        grid=(N_NODES // BM,),
        in_specs=[pl.BlockSpec((BM, D), lambda i: (i, 0)),
                  pl.BlockSpec((D, D), lambda i: (0, 0)),
                  pl.BlockSpec((D, D), lambda i: (0, 0)),
                  pl.BlockSpec((1, D), lambda i: (0, 0)),
                  pl.BlockSpec((BM, D), lambda i: (i, 0))],
        out_specs=pl.BlockSpec((BM, D), lambda i: (i, 0)),
        out_shape=jax.ShapeDtypeStruct((N_NODES, D), jnp.float32),
    )(x, W_self, W_nei, b_row, agg)


def _sc_edges(src3, dst3, w3, x_pad, zblk):
    mesh = plsc.VectorSubcoreMesh(core_axis_name="c", subcore_axis_name="s")

    @functools.partial(
        pl.kernel,
        mesh=mesh,
        out_type=jax.ShapeDtypeStruct((NC, HALF, D), jnp.float32),
        scratch_types=[
            pltpu.VMEM((2, G, CHUNK), jnp.int32),
            pltpu.VMEM((2, G, CHUNK), jnp.int32),
            pltpu.VMEM((2, G, CHUNK), jnp.float32),
            pltpu.VMEM((NBUF, CHUNK, D), jnp.float32),
            pltpu.VMEM_SHARED((X_ROWS, D), jnp.float32),
            pltpu.VMEM_SHARED((HALF, D), jnp.float32),
            pltpu.SemaphoreType.DMA,
            pltpu.SemaphoreType.DMA,
        ],
    )
    def k(src_hbm, dst_hbm, w_hbm, x_hbm, z_hbm, part_hbm,
          src_v, dst_v, w_v, rows_v, x_sp, acc, gsem, stgsem):
        c = lax.axis_index("c")
        s = lax.axis_index("s")
        sbase = s * SLAB
        xbase = s * XSLAB
        lo = c * HALF

        nfull = jnp.where(s == NS - 1, 32, 39)

        def xst(i, carry):
            @pl.when(i < nfull)
            def _():
                r0 = xbase + i * CHUNK
                pltpu.sync_copy(x_hbm.at[pl.ds(r0, CHUNK)], rows_v.at[0])
                pltpu.sync_copy(rows_v.at[0], x_sp.at[pl.ds(r0, CHUNK)])
            return carry
        lax.fori_loop(0, 39, xst, 0)
        r0t = xbase + nfull * CHUNK
        pltpu.sync_copy(x_hbm.at[pl.ds(r0t, 8)], rows_v.at[0].at[pl.ds(0, 8)])
        pltpu.sync_copy(rows_v.at[0].at[pl.ds(0, 8)], x_sp.at[pl.ds(r0t, 8)])

        pltpu.sync_copy(z_hbm, rows_v.at[1])

        def zb(i, carry):
            pltpu.sync_copy(rows_v.at[1], acc.at[pl.ds(sbase + i * CHUNK, CHUNK)])
            return carry
        lax.fori_loop(0, SLAB // CHUNK, zb, 0)
        plsc.subcore_barrier()

        def stage_start(gi, side):
            off = gi * G
            pltpu.async_copy(src_hbm.at[s].at[pl.ds(off, G)], src_v.at[side], stgsem)
            pltpu.async_copy(dst_hbm.at[s].at[pl.ds(off, G)], dst_v.at[side], stgsem)
            pltpu.async_copy(w_hbm.at[s].at[pl.ds(off, G)], w_v.at[side], stgsem)

        def stage_drain():
            pltpu.make_async_copy(src_hbm.at[s].at[pl.ds(0, G)], src_v.at[0], stgsem).wait()
            pltpu.make_async_copy(dst_hbm.at[s].at[pl.ds(0, G)], dst_v.at[0], stgsem).wait()
            pltpu.make_async_copy(w_hbm.at[s].at[pl.ds(0, G)], w_v.at[0], stgsem).wait()

        def transform(side):
            for j in range(G):
                for h in range(CHUNK // 16):
                    sl = pl.ds(h * 16, 16)
                    d16 = dst_v[side, j, sl]
                    m = jnp.logical_and(d16 >= lo, d16 < lo + HALF)
                    dst_v[side, j, sl] = jnp.where(m, d16 - lo, 0)
                    w16 = w_v[side, j, sl]
                    w_v[side, j, sl] = jnp.where(m, w16, 0.0)

        def gather_start(cg):
            side = lax.rem(cg // G, 2)
            j = lax.rem(cg, G)
            b = lax.rem(cg, NBUF)
            pltpu.async_copy(x_sp.at[src_v.at[side].at[j]], rows_v.at[b], gsem)

        stage_start(0, 0)
        stage_drain()
        transform(0)
        for p in range(NBUF - 1):
            gather_start(p)

        def chunk_body(ci, carry):
            b = lax.rem(ci, NBUF)
            gi = ci // G
            j = lax.rem(ci, G)
            side = lax.rem(gi, 2)

            @pl.when(jnp.logical_and(j == 0, gi + 1 < NGRP))
            def _():
                stage_start(gi + 1, lax.rem(gi + 1, 2))

            cg = ci + NBUF - 1
            @pl.when(cg < K)
            def _():
                @pl.when(lax.rem(cg, G) == 0)
                def _():
                    stage_drain()
                    transform(lax.rem(cg // G, 2))
                gather_start(cg)

            pltpu.make_async_copy(z_hbm, rows_v.at[b], gsem).wait()

            def grp(g16, ic):
                w16 = w_v[side, j, pl.ds(g16 * 16, 16)]
                for kk in range(16):
                    w = w16[kk]
                    e = g16 * 16 + kk
                    for g in range(D // 16):
                        fsl = pl.ds(g * 16, 16)
                        rows_v[b, e, fsl] = rows_v[b, e, fsl] * w
                return ic
            lax.fori_loop(0, CHUNK // 16, grp, 0)

            pltpu.sync_copy(rows_v.at[b], acc.at[dst_v.at[side].at[j]], add=True)
            return carry
        lax.fori_loop(0, K, chunk_body, 0)
        plsc.subcore_barrier()

        def wb(i, carry):
            r0 = sbase + i * CHUNK
            pltpu.sync_copy(acc.at[pl.ds(r0, CHUNK)], rows_v.at[0])
            pltpu.sync_copy(rows_v.at[0], part_hbm.at[c].at[pl.ds(r0, CHUNK)])
            return carry
        lax.fori_loop(0, SLAB // CHUNK, wb, 0)

    return k(src3, dst3, w3, x_pad, zblk)


def kernel(x, edge_index, edge_weight, W_self, b_self, W_nei):
    ei = edge_index.astype(jnp.int32)
    pad = E_PAD - N_EDGES
    src3 = jnp.pad(ei[0], (0, pad)).reshape(NS, K, CHUNK)
    dst3 = jnp.pad(ei[1], (0, pad)).reshape(NS, K, CHUNK)
    w3 = jnp.pad(edge_weight, (0, pad)).reshape(NS, K, CHUNK)
    x_pad = x
    zblk = jnp.zeros((CHUNK, D), jnp.float32)

    part = _sc_edges(src3, dst3, w3, x_pad, zblk)
    agg = part.reshape(NC * HALF, D)[:N_NODES]
    return _final(x, W_self, W_nei, b_self.reshape(1, D), agg)


# speedup vs baseline: 1.0003x; 1.0003x over previous
import functools

import jax
import jax.numpy as jnp
from jax import lax
from jax.experimental import pallas as pl
from jax.experimental.pallas import tpu as pltpu
from jax.experimental.pallas import tpu_sc as plsc

N_NODES = 10000
N_EDGES = 320000
D = 128

NC = 2
NS = 16
CHUNK = 16
G = 4
K = 1256
NGRP = K // G
NBUF = 2
E_PAD = NS * K * CHUNK

X_ROWS = 10000
XSLAB = 632
HALF = 5120
SLAB = HALF // NS
BM = 1000


def _self_body(x_ref, ws_ref, b_ref, h_ref):
    h_ref[...] = lax.dot_general(
        x_ref[...], ws_ref[...], (((1,), (1,)), ((), ())),
        preferred_element_type=jnp.float32) + b_ref[...]


def _self(x, W_self, b_row):
    return pl.pallas_call(
        _self_body,
        grid=(N_NODES // BM,),
        in_specs=[pl.BlockSpec((BM, D), lambda i: (i, 0)),
                  pl.BlockSpec((D, D), lambda i: (0, 0)),
                  pl.BlockSpec((1, D), lambda i: (0, 0))],
        out_specs=pl.BlockSpec((BM, D), lambda i: (i, 0)),
        out_shape=jax.ShapeDtypeStruct((N_NODES, D), jnp.float32),
    )(x, W_self, b_row)


def _final_body(h_ref, wn_ref, agg_ref, o_ref):
    nei = lax.dot_general(
        agg_ref[...], wn_ref[...], (((1,), (1,)), ((), ())),
        preferred_element_type=jnp.float32)
    o_ref[...] = h_ref[...] + nei


def _final(h, W_nei, agg):
    return pl.pallas_call(
        _final_body,
        grid=(N_NODES // BM,),
        in_specs=[pl.BlockSpec((BM, D), lambda i: (i, 0)),
                  pl.BlockSpec((D, D), lambda i: (0, 0)),
                  pl.BlockSpec((BM, D), lambda i: (i, 0))],
        out_specs=pl.BlockSpec((BM, D), lambda i: (i, 0)),
        out_shape=jax.ShapeDtypeStruct((N_NODES, D), jnp.float32),
    )(h, W_nei, agg)


def _sc_edges(src3, dst3, w3, x_pad, zblk):
    mesh = plsc.VectorSubcoreMesh(core_axis_name="c", subcore_axis_name="s")

    @functools.partial(
        pl.kernel,
        mesh=mesh,
        out_type=jax.ShapeDtypeStruct((NC, HALF, D), jnp.float32),
        scratch_types=[
            pltpu.VMEM((2, G, CHUNK), jnp.int32),
            pltpu.VMEM((2, G, CHUNK), jnp.int32),
            pltpu.VMEM((2, G, CHUNK), jnp.float32),
            pltpu.VMEM((NBUF, CHUNK, D), jnp.float32),
            pltpu.VMEM_SHARED((X_ROWS, D), jnp.float32),
            pltpu.VMEM_SHARED((HALF, D), jnp.float32),
            pltpu.SemaphoreType.DMA,
            pltpu.SemaphoreType.DMA,
        ],
    )
    def k(src_hbm, dst_hbm, w_hbm, x_hbm, z_hbm, part_hbm,
          src_v, dst_v, w_v, rows_v, x_sp, acc, gsem, stgsem):
        c = lax.axis_index("c")
        s = lax.axis_index("s")
        sbase = s * SLAB
        xbase = s * XSLAB
        lo = c * HALF

        nfull = jnp.where(s == NS - 1, 32, 39)

        def xst(i, carry):
            @pl.when(i < nfull)
            def _():
                r0 = xbase + i * CHUNK
                pltpu.sync_copy(x_hbm.at[pl.ds(r0, CHUNK)], rows_v.at[0])
                pltpu.sync_copy(rows_v.at[0], x_sp.at[pl.ds(r0, CHUNK)])
            return carry
        lax.fori_loop(0, 39, xst, 0)
        r0t = xbase + nfull * CHUNK
        pltpu.sync_copy(x_hbm.at[pl.ds(r0t, 8)], rows_v.at[0].at[pl.ds(0, 8)])
        pltpu.sync_copy(rows_v.at[0].at[pl.ds(0, 8)], x_sp.at[pl.ds(r0t, 8)])

        pltpu.sync_copy(z_hbm, rows_v.at[1])

        def zb(i, carry):
            pltpu.sync_copy(rows_v.at[1], acc.at[pl.ds(sbase + i * CHUNK, CHUNK)])
            return carry
        lax.fori_loop(0, SLAB // CHUNK, zb, 0)
        plsc.subcore_barrier()

        def stage_start(gi, side):
            off = gi * G
            pltpu.async_copy(src_hbm.at[s].at[pl.ds(off, G)], src_v.at[side], stgsem)
            pltpu.async_copy(dst_hbm.at[s].at[pl.ds(off, G)], dst_v.at[side], stgsem)
            pltpu.async_copy(w_hbm.at[s].at[pl.ds(off, G)], w_v.at[side], stgsem)

        def stage_drain():
            pltpu.make_async_copy(src_hbm.at[s].at[pl.ds(0, G)], src_v.at[0], stgsem).wait()
            pltpu.make_async_copy(dst_hbm.at[s].at[pl.ds(0, G)], dst_v.at[0], stgsem).wait()
            pltpu.make_async_copy(w_hbm.at[s].at[pl.ds(0, G)], w_v.at[0], stgsem).wait()

        def transform(side):
            for j in range(G):
                for h in range(CHUNK // 16):
                    sl = pl.ds(h * 16, 16)
                    d16 = dst_v[side, j, sl]
                    m = jnp.logical_and(d16 >= lo, d16 < lo + HALF)
                    dst_v[side, j, sl] = jnp.where(m, d16 - lo, 0)
                    w16 = w_v[side, j, sl]
                    w_v[side, j, sl] = jnp.where(m, w16, 0.0)

        def gather_start(cg):
            side = lax.rem(cg // G, 2)
            j = lax.rem(cg, G)
            b = lax.rem(cg, NBUF)
            pltpu.async_copy(x_sp.at[src_v.at[side].at[j]], rows_v.at[b], gsem)

        stage_start(0, 0)
        stage_drain()
        transform(0)
        for p in range(NBUF - 1):
            gather_start(p)

        def chunk_body(ci, carry):
            b = lax.rem(ci, NBUF)
            gi = ci // G
            j = lax.rem(ci, G)
            side = lax.rem(gi, 2)

            @pl.when(jnp.logical_and(j == 0, gi + 1 < NGRP))
            def _():
                stage_start(gi + 1, lax.rem(gi + 1, 2))

            cg = ci + NBUF - 1
            @pl.when(cg < K)
            def _():
                @pl.when(lax.rem(cg, G) == 0)
                def _():
                    stage_drain()
                    transform(lax.rem(cg // G, 2))
                gather_start(cg)

            pltpu.make_async_copy(z_hbm, rows_v.at[b], gsem).wait()

            def grp(g16, ic):
                w16 = w_v[side, j, pl.ds(g16 * 16, 16)]
                for kk in range(16):
                    w = w16[kk]
                    e = g16 * 16 + kk
                    for g in range(D // 16):
                        fsl = pl.ds(g * 16, 16)
                        rows_v[b, e, fsl] = rows_v[b, e, fsl] * w
                return ic
            lax.fori_loop(0, CHUNK // 16, grp, 0)

            pltpu.sync_copy(rows_v.at[b], acc.at[dst_v.at[side].at[j]], add=True)
            return carry
        lax.fori_loop(0, K, chunk_body, 0)
        plsc.subcore_barrier()

        def wb(i, carry):
            r0 = sbase + i * CHUNK
            pltpu.sync_copy(acc.at[pl.ds(r0, CHUNK)], rows_v.at[0])
            pltpu.sync_copy(rows_v.at[0], part_hbm.at[c].at[pl.ds(r0, CHUNK)])
            return carry
        lax.fori_loop(0, SLAB // CHUNK, wb, 0)

    return k(src3, dst3, w3, x_pad, zblk)


def kernel(x, edge_index, edge_weight, W_self, b_self, W_nei):
    ei = edge_index.astype(jnp.int32)
    pad = E_PAD - N_EDGES
    src3 = jnp.pad(ei[0], (0, pad)).reshape(NS, K, CHUNK)
    dst3 = jnp.pad(ei[1], (0, pad)).reshape(NS, K, CHUNK)
    w3 = jnp.pad(edge_weight, (0, pad)).reshape(NS, K, CHUNK)
    x_pad = x
    zblk = jnp.zeros((CHUNK, D), jnp.float32)

    h_self = _self(x, W_self, b_self.reshape(1, D))
    part = _sc_edges(src3, dst3, w3, x_pad, zblk)
    agg = part.reshape(NC * HALF, D)[:N_NODES]
    return _final(h_self, W_nei, agg)
